# baseline (speedup 1.0000x reference)
import functools

import jax
import jax.numpy as jnp
from jax import lax
from jax.experimental import pallas as pl
from jax.experimental.pallas import tpu as pltpu
from jax.experimental.pallas import tpu_sc as plsc

C = 8
V = 1024
H = 2048
B = 4
S = 2048
SCALE = 10.0

NW = 32
TOKENS = S * B
NBUF = 4

TC_TB = 512
TC_N = 3584
SC_N = TOKENS - TC_N


def _make_sc_body(t_per_w):
  idx_rows = t_per_w * C // 16
  nchunk = t_per_w

  def body(table_hbm, idx_hbm, out_hbm, idx_v, gbufs, obufs, gsems, osems):
    nc = 2
    wid = lax.axis_index("s") * nc + lax.axis_index("c")
    row0 = wid * idx_rows
    tok0 = wid * t_per_w

    pltpu.sync_copy(idx_hbm.at[pl.ds(row0, idx_rows)], idx_v)
    offs = (lax.iota(jnp.int32, 16) & 7) * V

    @pl.loop(0, idx_rows)
    def _(r):
      idx_v[r] = idx_v[r] + offs

    def idx_ref(chunk):
      half = (chunk & 1) * 8
      return idx_v.at[lax.shift_right_logical(chunk, 1), pl.ds(half, 8)]

    def start_gather(chunk, b):
      pltpu.async_copy(table_hbm.at[idx_ref(chunk)], gbufs.at[b], gsems.at[b])

    def wait_gather(b):
      pltpu.make_async_copy(
          table_hbm.at[idx_ref(0)], gbufs.at[b], gsems.at[b]).wait()

    for b in range(NBUF):
      start_gather(b, b)

    @pl.loop(0, nchunk, step=NBUF)
    def _(g):
      for b in range(NBUF):
        gc = g + b
        wait_gather(b)
        @pl.when(gc >= NBUF)
        def _():
          pltpu.make_async_copy(
              obufs.at[b], out_hbm.at[pl.ds(tok0, 1)], osems.at[b]).wait()

        gbuf = gbufs.at[b]
        obuf = obufs.at[b]

        @plsc.parallel_loop(0, H, 16, unroll=4)
        def _(j):
          col = pl.ds(j, 16)
          v = [gbuf[c, col] for c in range(C)]
          s01 = v[0] + v[1]
          s23 = v[2] + v[3]
          s45 = v[4] + v[5]
          s67 = v[6] + v[7]
          obuf[0, col] = ((s01 + s23) + (s45 + s67)) * SCALE

        pltpu.async_copy(obufs.at[b], out_hbm.at[pl.ds(tok0 + gc, 1)],
                         osems.at[b])

        @pl.when(gc + NBUF < nchunk)
        def _():
          start_gather(gc + NBUF, b)

    for b in range(NBUF):
      pltpu.make_async_copy(obufs.at[b], out_hbm.at[pl.ds(tok0, 1)],
                            osems.at[b]).wait()

  return body, idx_rows


def _run_sc(table_flat, idx2d, n_tok):
  t_per_w = n_tok // NW
  body, idx_rows = _make_sc_body(t_per_w)
  mesh = plsc.VectorSubcoreMesh(core_axis_name="c", subcore_axis_name="s")
  return pl.kernel(
      body,
      out_type=jax.ShapeDtypeStruct((n_tok, H), jnp.float32),
      mesh=mesh,
      scratch_types=[
          pltpu.VMEM((idx_rows, 16), jnp.int32),
          pltpu.VMEM((NBUF, C, H), jnp.float32),
          pltpu.VMEM((NBUF, 1, H), jnp.float32),
          pltpu.SemaphoreType.DMA((NBUF,)),
          pltpu.SemaphoreType.DMA((NBUF,)),
      ],
  )(table_flat, idx2d)


def _tc_body(ids_ref, table_ref, out_ref):
  ids = ids_ref[...]
  cols = []
  for c in range(C):
    iota = lax.broadcasted_iota(jnp.int32, (TC_TB, V), 1)
    onehot = (iota == ids[:, c][:, None]).astype(jnp.bfloat16)
    cols.append(onehot)
  onehot_full = jnp.concatenate(cols, axis=1)
  acc = jax.lax.dot_general(
      onehot_full, table_ref[...],
      (((1,), (0,)), ((), ())),
      preferred_element_type=jnp.float32)
  out_ref[...] = acc * SCALE


def _run_tc(ids_tok, table_bf16, n_tok):
  grid = (n_tok // TC_TB,)
  return pl.pallas_call(
      _tc_body,
      grid=grid,
      in_specs=[
          pl.BlockSpec((TC_TB, C), lambda i: (i, 0)),
          pl.BlockSpec((C * V, H), lambda i: (0, 0)),
      ],
      out_specs=pl.BlockSpec((TC_TB, H), lambda i: (i, 0)),
      out_shape=jax.ShapeDtypeStruct((n_tok, H), jnp.float32),
  )(ids_tok, table_bf16)


@jax.jit
def _run_hybrid(table_flat, table_bf16, ids_tok_tc, idx2d_sc):
  out_sc = _run_sc(table_flat, idx2d_sc, SC_N)
  ids_dep = ids_tok_tc + (out_sc[0, 0] * 0.0).astype(jnp.int32)
  out_tc = _run_tc(ids_dep, table_bf16, TC_N)
  return jnp.concatenate([out_tc, out_sc], axis=0)


@jax.jit
def _run_sc_all(table_flat, idx2d):
  return _run_sc(table_flat, idx2d, TOKENS)


@jax.jit
def _run_sc_split(table_flat, idx2d_head, idx2d_tail):
  a = _run_sc(table_flat, idx2d_head, TC_N)
  b = _run_sc(table_flat, idx2d_tail, SC_N)
  return jnp.concatenate([a, b], axis=0)


def kernel(audio_ids, tables):
  ids = jnp.transpose(audio_ids, (1, 0, 2))
  ids_tok = ids.reshape(TOKENS, C)
  table_flat = tables.reshape(C * V, H)
  table_bf16 = table_flat.astype(jnp.bfloat16)
  idx2d = ids.reshape(TOKENS * C // 16, 16)
  out = _run_hybrid(table_flat, table_bf16,
                    ids_tok[:TC_N], idx2d[TC_N * C // 16:])
  return out.reshape(S, B, H)


# speedup vs baseline: 1.8629x; 1.8629x over previous
import functools

import jax
import jax.numpy as jnp
from jax import lax
from jax.experimental import pallas as pl
from jax.experimental.pallas import tpu as pltpu
from jax.experimental.pallas import tpu_sc as plsc

C = 8
V = 1024
H = 2048
B = 4
S = 2048
SCALE = 10.0

NW = 32
TOKENS = S * B
T_PER_W = TOKENS // NW
NBUF = 4
NCHUNK = T_PER_W
IDX_ROWS = T_PER_W * C // 16


def _body(table_hbm, idx_hbm, out_hbm,
          idx_v, gbufs, obufs, gsems, osems):
  nc = 2
  wid = lax.axis_index("s") * nc + lax.axis_index("c")
  row0 = wid * IDX_ROWS
  tok0 = wid * T_PER_W

  pltpu.sync_copy(idx_hbm.at[pl.ds(row0, IDX_ROWS)], idx_v)
  offs = (lax.iota(jnp.int32, 16) & 7) * V

  @pl.loop(0, IDX_ROWS)
  def _(r):
    idx_v[r] = idx_v[r] + offs

  def idx_ref(chunk):
    half = (chunk & 1) * 8
    return idx_v.at[lax.shift_right_logical(chunk, 1), pl.ds(half, 8)]

  def start_gather(chunk, b):
    pltpu.async_copy(table_hbm.at[idx_ref(chunk)], gbufs.at[b], gsems.at[b])

  def wait_gather(b):
    pltpu.make_async_copy(
        table_hbm.at[idx_ref(0)], gbufs.at[b], gsems.at[b]).wait()

  for b in range(NBUF):
    start_gather(b, b)

  @pl.loop(0, NCHUNK, step=NBUF)
  def _(g):
    for b in range(NBUF):
      gc = g + b
      wait_gather(b)
      @pl.when(gc >= NBUF)
      def _():
        pltpu.make_async_copy(
            obufs.at[b, 0], out_hbm.at[0, 0], osems.at[b]).wait()

      gbuf = gbufs.at[b]
      obuf = obufs.at[b]

      @plsc.parallel_loop(0, H, 16, unroll=4)
      def _(j):
        col = pl.ds(j, 16)
        v = [gbuf[c, col] for c in range(C)]
        s01 = v[0] + v[1]
        s23 = v[2] + v[3]
        s45 = v[4] + v[5]
        s67 = v[6] + v[7]
        obuf[0, col] = ((s01 + s23) + (s45 + s67)) * SCALE

      tok = tok0 + gc
      pltpu.async_copy(obufs.at[b, 0],
                       out_hbm.at[lax.shift_right_logical(tok, 2), tok & 3],
                       osems.at[b])

      @pl.when(gc + NBUF < NCHUNK)
      def _():
        start_gather(gc + NBUF, b)

  for b in range(NBUF):
    pltpu.make_async_copy(obufs.at[b, 0], out_hbm.at[0, 0],
                          osems.at[b]).wait()


@jax.jit
def _run(table_flat, idx2d):
  mesh = plsc.VectorSubcoreMesh(core_axis_name="c", subcore_axis_name="s")
  return pl.kernel(
      _body,
      out_type=jax.ShapeDtypeStruct((S, B, H), jnp.float32),
      mesh=mesh,
      scratch_types=[
          pltpu.VMEM((IDX_ROWS, 16), jnp.int32),
          pltpu.VMEM((NBUF, C, H), jnp.float32),
          pltpu.VMEM((NBUF, 1, H), jnp.float32),
          pltpu.SemaphoreType.DMA((NBUF,)),
          pltpu.SemaphoreType.DMA((NBUF,)),
      ],
  )(table_flat, idx2d)


def kernel(audio_ids, tables):
  ids = jnp.transpose(audio_ids, (1, 0, 2))
  idx2d = ids.reshape(TOKENS * C // 16, 16)
  table_flat = tables.reshape(C * V, H)
  return _run(table_flat, idx2d)
